# baseline (speedup 1.0000x reference)
import jax
import jax.numpy as jnp
from jax.experimental import pallas as pl
from jax.experimental.pallas import tpu as pltpu
from jax.experimental.pallas import tpu_sc as plsc

B, T, D = 4, 8192, 768
E = 8
TOP_K = 2
CITY_DIM = 32

BLK = 2048
CHK = 512
LANES = 16

NEG = -1e30


def _logits_body(mh_ref, dt_ref, dd_ref, rg_ref, de_ref, w_ref, cb_ref, lt_ref):
    acc = jnp.dot(mh_ref[0], jnp.transpose(w_ref[:, 0:768]),
                  preferred_element_type=jnp.float32)
    lt = jnp.transpose(acc)
    lt += jnp.dot(w_ref[:, 800:992], dt_ref[0],
                  preferred_element_type=jnp.float32)
    lt += jnp.dot(w_ref[:, 992:1184], dd_ref[0],
                  preferred_element_type=jnp.float32)
    lt += jnp.dot(w_ref[:, 1184:1280], rg_ref[0],
                  preferred_element_type=jnp.float32)
    lt += jnp.dot(w_ref[:, 1280:1376], de_ref[0],
                  preferred_element_type=jnp.float32)
    lt_ref[...] = lt + cb_ref[...]


def _run_logits(mh, dtT, ddT, rgT, deT, w, cb):
    grid = (B, T // BLK)
    tok = lambda b, i: (b, i, 0)
    feat = lambda b, i: (b, 0, i)
    fixed = lambda b, i: (0, 0)
    return pl.pallas_call(
        _logits_body,
        grid=grid,
        in_specs=[
            pl.BlockSpec((1, BLK, D), tok),
            pl.BlockSpec((1, D // 4, BLK), feat),
            pl.BlockSpec((1, D // 4, BLK), feat),
            pl.BlockSpec((1, D // 8, BLK), feat),
            pl.BlockSpec((1, D // 8, BLK), feat),
            pl.BlockSpec(w.shape, fixed),
            pl.BlockSpec(cb.shape, fixed),
        ],
        out_specs=pl.BlockSpec((E, BLK), lambda b, i: (b, i)),
        out_shape=jax.ShapeDtypeStruct((B * E, T), jnp.float32),
        compiler_params=pltpu.CompilerParams(
            dimension_semantics=("arbitrary", "arbitrary"),
        ),
    )(mh, dtT, ddT, rgT, deT, w, cb)


def _route_block(lg_vmem, r_vmem, i_vmem, g_vmem):
    @pl.loop(0, CHK, step=LANES)
    def _(c):
        sl = lambda e: (pl.ds(e, 1), pl.ds(c, LANES))
        l = [lg_vmem.at[*sl(e)][...] for e in range(E)]

        m1 = l[0]
        for e in range(1, E):
            m1 = jnp.maximum(m1, l[e])
        idx1 = jnp.full((1, LANES), float(E - 1), jnp.float32)
        for e in range(E - 2, -1, -1):
            idx1 = jnp.where(l[e] == m1, float(e), idx1)

        negv = jnp.full((1, LANES), NEG, jnp.float32)
        l2 = [jnp.where(idx1 == float(e), negv, l[e]) for e in range(E)]
        m2 = l2[0]
        for e in range(1, E):
            m2 = jnp.maximum(m2, l2[e])
        idx2 = jnp.full((1, LANES), float(E - 1), jnp.float32)
        for e in range(E - 2, -1, -1):
            idx2 = jnp.where(l2[e] == m2, float(e), idx2)

        ex = [jnp.exp(l[e] - m1) for e in range(E)]
        s = ex[0]
        for e in range(1, E):
            s = s + ex[e]
        rs = 1.0 / s

        e2 = jnp.exp(m2 - m1)
        p1 = 1.0 / (1.0 + e2)
        p2 = 1.0 - p1
        zero = jnp.zeros((1, LANES), jnp.float32)
        for e in range(E):
            g_vmem.at[*sl(e)][...] = ex[e] * rs
            r_vmem.at[*sl(e)][...] = jnp.where(
                idx1 == float(e), p1, jnp.where(idx2 == float(e), p2, zero))
        i_vmem.at[*sl(0)][...] = idx1.astype(jnp.int32)
        i_vmem.at[*sl(1)][...] = idx2.astype(jnp.int32)


def _sc_route(lg):
    mesh = plsc.VectorSubcoreMesh(core_axis_name="c", subcore_axis_name="s")

    @pl.kernel(
        out_type=[
            jax.ShapeDtypeStruct((B * E, T), jnp.float32),
            jax.ShapeDtypeStruct((B * TOP_K, T), jnp.int32),
            jax.ShapeDtypeStruct((B * E, T), jnp.float32),
        ],
        mesh=mesh,
    )
    def sck(lg_hbm, r_hbm, i_hbm, g_hbm):
        pltpu.emit_pipeline(
            _route_block,
            grid=(B, T // CHK),
            in_specs=[pl.BlockSpec((E, CHK), lambda b, c: (b, c))],
            out_specs=[
                pl.BlockSpec((E, CHK), lambda b, c: (b, c)),
                pl.BlockSpec((TOP_K, CHK), lambda b, c: (b, c)),
                pl.BlockSpec((E, CHK), lambda b, c: (b, c)),
            ],
            core_axis_name=("c", "s"),
            dimension_semantics=(pltpu.PARALLEL, pltpu.PARALLEL),
        )(lg_hbm, r_hbm, i_hbm, g_hbm)

    return sck(lg)


@jax.jit
def _run(mh, dtT, ddT, rgT, deT, w, cb):
    lg = _run_logits(mh, dtT, ddT, rgT, deT, w, cb)
    routerF, idxF, gateF = _sc_route(lg)
    swap = lambda a: jnp.transpose(a, (0, 2, 1))
    return (swap(routerF.reshape(B, E, T)),
            swap(idxF.reshape(B, TOP_K, T)),
            swap(gateF.reshape(B, E, T)))


def kernel(mh_output, delta_t_info, delta_dis_info, delta_rg_info,
           delta_entropy_info, city_embeddings, W_topk, b_topk, city):
    cb = (city_embeddings[city] @ W_topk[D:D + CITY_DIM, :] + b_topk).reshape(E, 1)
    swap = lambda a: jnp.transpose(a, (0, 2, 1))
    return _run(mh_output, swap(delta_t_info), swap(delta_dis_info),
                swap(delta_rg_info), swap(delta_entropy_info),
                jnp.transpose(W_topk), cb)


# speedup vs baseline: 1.4302x; 1.4302x over previous
import jax
import jax.numpy as jnp
from jax.experimental import pallas as pl
from jax.experimental.pallas import tpu as pltpu

B, T, D = 4, 8192, 768
E = 8
TOP_K = 2
CITY_DIM = 32

BLK = 2048

NEG_INF = float("-inf")


def _router_body(mh_ref, dt_ref, dd_ref, rg_ref, de_ref, w_ref, crow_ref,
                 b_ref, router_ref, idx_ref, gate_ref):
    acc = jnp.dot(mh_ref[0], jnp.transpose(w_ref[:, 0:768]),
                  preferred_element_type=jnp.float32)
    lt = jnp.transpose(acc)
    lt += jnp.dot(w_ref[:, 800:992], dt_ref[0],
                  preferred_element_type=jnp.float32)
    lt += jnp.dot(w_ref[:, 992:1184], dd_ref[0],
                  preferred_element_type=jnp.float32)
    lt += jnp.dot(w_ref[:, 1184:1280], rg_ref[0],
                  preferred_element_type=jnp.float32)
    lt += jnp.dot(w_ref[:, 1280:1376], de_ref[0],
                  preferred_element_type=jnp.float32)
    cb = jnp.dot(w_ref[:, 768:800], crow_ref[...],
                 preferred_element_type=jnp.float32)
    lt += cb + b_ref[...]

    srow = jax.lax.broadcasted_iota(jnp.int32, lt.shape, 0).astype(jnp.float32)

    m1 = jnp.max(lt, axis=0, keepdims=True)
    ex = jnp.exp(lt - m1)
    gate_ref[...] = (ex / jnp.sum(ex, axis=0, keepdims=True))[None]

    idx1 = jnp.min(jnp.where(lt == m1, srow, float(E)), axis=0, keepdims=True)
    is1 = srow == idx1
    l2 = jnp.where(is1, NEG_INF, lt)
    m2 = jnp.max(l2, axis=0, keepdims=True)
    idx2 = jnp.min(jnp.where(l2 == m2, srow, float(E)), axis=0, keepdims=True)
    is2 = srow == idx2

    e2 = jnp.exp(m2 - m1)
    denom = 1.0 + e2
    p1 = jnp.broadcast_to(1.0 / denom, lt.shape)
    p2 = jnp.broadcast_to(e2 / denom, lt.shape)
    router_ref[...] = jnp.where(is1, p1, jnp.where(is2, p2, 0.0))[None]

    idx_ref[...] = jnp.concatenate([idx1, idx2], axis=0).astype(jnp.int32)[None]


@jax.jit
def _run(mh, dtT, ddT, rgT, deT, w, crow, b):
    grid = (B, T // BLK)
    tok = lambda b, i: (b, i, 0)
    feat = lambda b, i: (b, 0, i)
    fixed = lambda b, i: (0, 0)
    out = pl.pallas_call(
        _router_body,
        grid=grid,
        in_specs=[
            pl.BlockSpec((1, BLK, D), tok),
            pl.BlockSpec((1, D // 4, BLK), feat),
            pl.BlockSpec((1, D // 4, BLK), feat),
            pl.BlockSpec((1, D // 8, BLK), feat),
            pl.BlockSpec((1, D // 8, BLK), feat),
            pl.BlockSpec(w.shape, fixed),
            pl.BlockSpec(crow.shape, fixed),
            pl.BlockSpec(b.shape, fixed),
        ],
        out_specs=[
            pl.BlockSpec((1, E, BLK), feat),
            pl.BlockSpec((1, TOP_K, BLK), feat),
            pl.BlockSpec((1, E, BLK), feat),
        ],
        out_shape=[
            jax.ShapeDtypeStruct((B, E, T), jnp.float32),
            jax.ShapeDtypeStruct((B, TOP_K, T), jnp.int32),
            jax.ShapeDtypeStruct((B, E, T), jnp.float32),
        ],
        compiler_params=pltpu.CompilerParams(
            dimension_semantics=("arbitrary", "arbitrary"),
        ),
    )(mh, dtT, ddT, rgT, deT, w, crow, b)
    return out


def kernel(mh_output, delta_t_info, delta_dis_info, delta_rg_info,
           delta_entropy_info, city_embeddings, W_topk, b_topk, city):
    crow = city_embeddings[city].reshape(CITY_DIM, 1)
    swap = lambda a: jnp.transpose(a, (0, 2, 1))
    routerT, idxT, gateT = _run(
        mh_output, swap(delta_t_info), swap(delta_dis_info),
        swap(delta_rg_info), swap(delta_entropy_info), jnp.transpose(W_topk),
        crow, b_topk.reshape(E, 1))
    return (swap(routerT), swap(idxT), swap(gateT))
